# baseline (speedup 1.0000x reference)
import functools

import jax
import jax.numpy as jnp
from jax import lax
from jax.experimental import pallas as pl
from jax.experimental.pallas import tpu as pltpu
from jax.experimental.pallas import tpu_sc as plsc

N = 10000
E = 320000
D = 128
HD = D // 2
CHUNK = 125
CPT = E // (16 * CHUNK)
ROWS_PT = N // 16
RCH = 125
NRC = 5
NBUF = 4


def _segsum_sc(src2, dst2, xs):
    mesh = plsc.VectorSubcoreMesh(core_axis_name="c", subcore_axis_name="s")

    @functools.partial(
        pl.kernel,
        out_type=jax.ShapeDtypeStruct((2, N, HD), jnp.float32),
        mesh=mesh,
        compiler_params=pltpu.CompilerParams(use_tc_tiling_on_sc=False),
        scratch_types=[
            pltpu.VMEM((CPT, CHUNK), jnp.int32),
            pltpu.VMEM((CPT, CHUNK), jnp.int32),
            *[pltpu.VMEM((CHUNK, HD), jnp.float32) for _ in range(NBUF)],
            pltpu.VMEM_SHARED((N, HD), jnp.float32),
            *[pltpu.SemaphoreType.DMA for _ in range(2 * NBUF)],
        ],
    )
    def k(src_hbm, dst_hbm, x_hbm, out_hbm, src_v, dst_v, *rest):
        rows = list(rest[:NBUF])
        agg_sh = rest[NBUF]
        gsem = list(rest[NBUF + 1:2 * NBUF + 1])
        ssem = list(rest[2 * NBUF + 1:])
        c = lax.axis_index("c")
        s = lax.axis_index("s")

        zero = jnp.zeros((16,), jnp.float32)

        def zrow(i, carry):
            for j in range(HD // 16):
                rows[0][i, pl.ds(j * 16, 16)] = zero
            return carry

        lax.fori_loop(0, RCH, zrow, 0)
        for t in range(NRC):
            pltpu.sync_copy(rows[0],
                            agg_sh.at[pl.ds(s * ROWS_PT + t * RCH, RCH)])
        plsc.subcore_barrier()

        row0 = s * CPT
        pltpu.sync_copy(src_hbm.at[pl.ds(row0, CPT)], src_v)
        pltpu.sync_copy(dst_hbm.at[pl.ds(row0, CPT)], dst_v)

        HB = NBUF // 2

        def drain(buf, sem):
            pltpu.make_async_copy(x_hbm.at[0].at[pl.ds(0, CHUNK)], buf,
                                  sem).wait()

        def gather(j, b):
            pltpu.async_copy(x_hbm.at[c].at[src_v.at[j]], rows[b], gsem[b])

        def scatter(j, b):
            pltpu.async_copy(rows[b], agg_sh.at[dst_v.at[j]], ssem[b],
                             add=True)

        for b in range(HB):
            gather(b, b)
        for b in range(NBUF):
            drain(rows[b], gsem[b])
            scatter(b, b)
            bn = (b + HB) % NBUF
            if b >= HB:
                drain(rows[bn], ssem[bn])
            gather(b + HB, bn)

        def step(i, carry):
            for b in range(NBUF):
                j = i * NBUF + b
                bn = (b + HB) % NBUF
                drain(rows[b], gsem[b])
                scatter(j, b)
                drain(rows[bn], ssem[bn])
                pltpu.async_copy(
                    x_hbm.at[c].at[src_v.at[jnp.minimum(j + HB, CPT - 1)]],
                    rows[bn], gsem[bn])
            return carry

        lax.fori_loop(1, CPT // NBUF, step, 0)
        for b in range(NBUF):
            if b < HB:
                drain(rows[b], gsem[b])
            else:
                drain(rows[b], ssem[b])
        plsc.subcore_barrier()

        for t in range(NRC):
            r0 = s * ROWS_PT + t * RCH
            b = t % 2
            if t >= 2:
                drain(rows[b], gsem[b])
            pltpu.sync_copy(agg_sh.at[pl.ds(r0, RCH)], rows[b])
            pltpu.async_copy(rows[b], out_hbm.at[c].at[pl.ds(r0, RCH)],
                             gsem[b])
        for b in range(2):
            drain(rows[b], gsem[b])

    return k(src2, dst2, xs)


def _make_dense_body(relu, residual, split_out):
    def body(p_ref, x_ref, wr_ref, wo_ref, b_ref, o_ref):
        out = jnp.dot(p_ref[0], wr_ref[0], preferred_element_type=jnp.float32)
        out = out + jnp.dot(p_ref[1], wr_ref[1],
                            preferred_element_type=jnp.float32)
        out = out + jnp.dot(x_ref[0], wo_ref[0],
                            preferred_element_type=jnp.float32)
        out = out + jnp.dot(x_ref[1], wo_ref[1],
                            preferred_element_type=jnp.float32)
        out = out + b_ref[...]
        if relu:
            out = jnp.maximum(out, 0.0)
        o0 = out[:, :HD]
        o1 = out[:, HD:]
        if residual:
            o0 = o0 + x_ref[0]
            o1 = o1 + x_ref[1]
        if split_out:
            o_ref[0] = o0
            o_ref[1] = o1
        else:
            o_ref[...] = jnp.concatenate([o0, o1], axis=1)
    return body


def _dense(p, xs, wr, wo, b, relu, residual, split_out):
    BR = 1000
    if split_out:
        out_shape = jax.ShapeDtypeStruct((2, N, HD), jnp.float32)
        out_spec = pl.BlockSpec((2, BR, HD), lambda i: (0, i, 0))
    else:
        out_shape = jax.ShapeDtypeStruct((N, D), jnp.float32)
        out_spec = pl.BlockSpec((BR, D), lambda i: (i, 0))
    return pl.pallas_call(
        _make_dense_body(relu, residual, split_out),
        grid=(N // BR,),
        in_specs=[
            pl.BlockSpec((2, BR, HD), lambda i: (0, i, 0)),
            pl.BlockSpec((2, BR, HD), lambda i: (0, i, 0)),
            pl.BlockSpec((2, HD, D), lambda i: (0, 0, 0)),
            pl.BlockSpec((2, HD, D), lambda i: (0, 0, 0)),
            pl.BlockSpec((1, D), lambda i: (0, 0)),
        ],
        out_specs=out_spec,
        out_shape=out_shape,
    )(p, xs, wr, wo, b)


def kernel(x, edge_index,
           W_rel1, W_root1, b1,
           W_rel2, W_root2, b2,
           W_rel3, W_root3, b3,
           W_rel4, W_root4, b4):
    ei = edge_index.astype(jnp.int32)
    src2 = ei[0].reshape(E // CHUNK, CHUNK)
    dst2 = ei[1].reshape(E // CHUNK, CHUNK)

    xs = jnp.stack([x[:, :HD], x[:, HD:]])

    def wsplit(w):
        return jnp.stack([w[:HD, :], w[HD:, :]])

    def layer(xs_in, wr, wo, b, relu, residual, split_out):
        p = _segsum_sc(src2, dst2, xs_in)
        return _dense(p, xs_in, wsplit(wr), wsplit(wo), b.reshape(1, D),
                      relu, residual, split_out)

    x1 = layer(xs, W_rel1, W_root1, b1, True, False, True)
    x2 = layer(x1, W_rel2, W_root2, b2, True, True, True)
    x3 = layer(x2, W_rel3, W_root3, b3, False, True, True)
    x4 = layer(x3, W_rel4, W_root4, b4, False, True, False)
    return x4


# speedup vs baseline: 1.1932x; 1.1932x over previous
import functools

import jax
import jax.numpy as jnp
from jax import lax
from jax.experimental import pallas as pl
from jax.experimental.pallas import tpu as pltpu
from jax.experimental.pallas import tpu_sc as plsc

N = 10000
E = 320000
D = 128
HD = D // 2
CHUNK = 125
CPT = E // (16 * CHUNK)
ROWS_PT = N // 16
RCH = 125
NRC = 5
NBUF = 4


def _segsum_sc(srcp, dst2, xv):
    mesh = plsc.VectorSubcoreMesh(core_axis_name="c", subcore_axis_name="s")

    @functools.partial(
        pl.kernel,
        out_type=jax.ShapeDtypeStruct((N, D), jnp.float32),
        mesh=mesh,
        compiler_params=pltpu.CompilerParams(use_tc_tiling_on_sc=False),
        scratch_types=[
            pltpu.VMEM((CPT, CHUNK), jnp.int32),
            pltpu.VMEM((CPT, CHUNK), jnp.int32),
            *[pltpu.VMEM((CHUNK, HD), jnp.float32) for _ in range(NBUF)],
            pltpu.VMEM_SHARED((N, HD), jnp.float32),
            *[pltpu.SemaphoreType.DMA for _ in range(2 * NBUF)],
        ],
    )
    def k(src_hbm, dst_hbm, x_hbm, out_hbm, src_v, dst_v, *rest):
        rows = list(rest[:NBUF])
        agg_sh = rest[NBUF]
        gsem = list(rest[NBUF + 1:2 * NBUF + 1])
        ssem = list(rest[2 * NBUF + 1:])
        c = lax.axis_index("c")
        s = lax.axis_index("s")
        col0 = c * HD

        zero = jnp.zeros((16,), jnp.float32)

        def zrow(i, carry):
            for j in range(HD // 16):
                rows[0][i, pl.ds(j * 16, 16)] = zero
            return carry

        lax.fori_loop(0, RCH, zrow, 0)
        for t in range(NRC):
            pltpu.sync_copy(rows[0],
                            agg_sh.at[pl.ds(s * ROWS_PT + t * RCH, RCH)])
        plsc.subcore_barrier()

        row0 = s * CPT
        pltpu.sync_copy(src_hbm.at[c, pl.ds(row0, CPT)], src_v)
        pltpu.sync_copy(dst_hbm.at[pl.ds(row0, CPT)], dst_v)

        HB = NBUF // 2

        def drain(buf, sem):
            pltpu.make_async_copy(x_hbm.at[pl.ds(0, CHUNK)], buf, sem).wait()

        def gather(j, b):
            pltpu.async_copy(x_hbm.at[src_v.at[j]], rows[b], gsem[b])

        def scatter(j, b):
            pltpu.async_copy(rows[b], agg_sh.at[dst_v.at[j]], ssem[b],
                             add=True)

        for b in range(HB):
            gather(b, b)
        for b in range(NBUF):
            drain(rows[b], gsem[b])
            scatter(b, b)
            bn = (b + HB) % NBUF
            if b >= HB:
                drain(rows[bn], ssem[bn])
            gather(b + HB, bn)

        def step(i, carry):
            for b in range(NBUF):
                j = i * NBUF + b
                bn = (b + HB) % NBUF
                drain(rows[b], gsem[b])
                scatter(j, b)
                drain(rows[bn], ssem[bn])
                gather(jnp.minimum(j + HB, CPT - 1), bn)
            return carry

        lax.fori_loop(1, CPT // NBUF, step, 0)
        for b in range(NBUF):
            if b < HB:
                drain(rows[b], gsem[b])
            else:
                drain(rows[b], ssem[b])
        plsc.subcore_barrier()

        for t in range(NRC):
            r0 = s * ROWS_PT + t * RCH
            b = t % 2
            if t >= 2:
                drain(rows[b], gsem[b])
            pltpu.sync_copy(agg_sh.at[pl.ds(r0, RCH)], rows[b])
            pltpu.async_copy(rows[b],
                             out_hbm.at[pl.ds(r0, RCH), pl.ds(col0, HD)],
                             gsem[b])
        for b in range(2):
            drain(rows[b], gsem[b])

    return k(srcp, dst2, xv)


def _make_dense_body(relu, residual):
    def body(p_ref, x_ref, wr_ref, wo_ref, b_ref, o_ref):
        out = jnp.dot(p_ref[...], wr_ref[...],
                      preferred_element_type=jnp.float32)
        out = out + jnp.dot(x_ref[...], wo_ref[...],
                            preferred_element_type=jnp.float32)
        out = out + b_ref[...]
        if relu:
            out = jnp.maximum(out, 0.0)
        if residual:
            out = out + x_ref[...]
        o_ref[...] = out
    return body


def _dense(p, xin, wr, wo, b, relu, residual):
    BR = 1000
    return pl.pallas_call(
        _make_dense_body(relu, residual),
        grid=(N // BR,),
        in_specs=[
            pl.BlockSpec((BR, D), lambda i: (i, 0)),
            pl.BlockSpec((BR, D), lambda i: (i, 0)),
            pl.BlockSpec((D, D), lambda i: (0, 0)),
            pl.BlockSpec((D, D), lambda i: (0, 0)),
            pl.BlockSpec((1, D), lambda i: (0, 0)),
        ],
        out_specs=pl.BlockSpec((BR, D), lambda i: (i, 0)),
        out_shape=jax.ShapeDtypeStruct((N, D), jnp.float32),
    )(p, xin, wr, wo, b)


def kernel(x, edge_index,
           W_rel1, W_root1, b1,
           W_rel2, W_root2, b2,
           W_rel3, W_root3, b3,
           W_rel4, W_root4, b4):
    ei = edge_index.astype(jnp.int32)
    src = ei[0].reshape(E // CHUNK, CHUNK)
    dst2 = ei[1].reshape(E // CHUNK, CHUNK)
    srcp = jnp.stack([2 * src, 2 * src + 1])

    def layer(xin, wr, wo, b, relu, residual):
        p = _segsum_sc(srcp, dst2, xin.reshape(2 * N, HD))
        return _dense(p, xin, wr, wo, b.reshape(1, D), relu, residual)

    x1 = layer(x, W_rel1, W_root1, b1, True, False)
    x2 = layer(x1, W_rel2, W_root2, b2, True, True)
    x3 = layer(x2, W_rel3, W_root3, b3, False, True)
    x4 = layer(x3, W_rel4, W_root4, b4, False, True)
    return x4


# speedup vs baseline: 1.2004x; 1.0061x over previous
import functools

import jax
import jax.numpy as jnp
from jax import lax
from jax.experimental import pallas as pl
from jax.experimental.pallas import tpu as pltpu
from jax.experimental.pallas import tpu_sc as plsc

N = 10000
E = 320000
D = 128
HD = D // 2
CHUNK = 125
CPT = E // (16 * CHUNK)
ROWS_PT = N // 16
RCH = 125
NRC = 5
NBUF = 4


def _segsum_sc(srcp, dst2, xv):
    mesh = plsc.VectorSubcoreMesh(core_axis_name="c", subcore_axis_name="s")

    @functools.partial(
        pl.kernel,
        out_type=jax.ShapeDtypeStruct((N, D), jnp.float32),
        mesh=mesh,
        compiler_params=pltpu.CompilerParams(use_tc_tiling_on_sc=False),
        scratch_types=[
            pltpu.VMEM((CPT, CHUNK), jnp.int32),
            pltpu.VMEM((CPT, CHUNK), jnp.int32),
            *[pltpu.VMEM((CHUNK, HD), jnp.float32) for _ in range(NBUF)],
            pltpu.VMEM_SHARED((N, HD), jnp.float32),
            *[pltpu.SemaphoreType.DMA for _ in range(2 * NBUF)],
        ],
    )
    def k(src_hbm, dst_hbm, x_hbm, out_hbm, src_v, dst_v, *rest):
        rows = list(rest[:NBUF])
        agg_sh = rest[NBUF]
        gsem = list(rest[NBUF + 1:2 * NBUF + 1])
        ssem = list(rest[2 * NBUF + 1:])
        c = lax.axis_index("c")
        s = lax.axis_index("s")
        col0 = c * HD

        zero = jnp.zeros((16,), jnp.float32)

        def zrow(i, carry):
            for j in range(HD // 16):
                rows[0][i, pl.ds(j * 16, 16)] = zero
            return carry

        lax.fori_loop(0, RCH, zrow, 0)
        for t in range(NRC):
            pltpu.sync_copy(rows[0],
                            agg_sh.at[pl.ds(s * ROWS_PT + t * RCH, RCH)])
        plsc.subcore_barrier()

        row0 = s * CPT
        pltpu.sync_copy(src_hbm.at[c, pl.ds(row0, CPT)], src_v)
        pltpu.sync_copy(dst_hbm.at[pl.ds(row0, CPT)], dst_v)

        HB = NBUF // 2

        def drain(buf, sem):
            pltpu.make_async_copy(x_hbm.at[pl.ds(0, CHUNK)], buf, sem).wait()

        def gather(j, b):
            pltpu.async_copy(x_hbm.at[src_v.at[j]], rows[b], gsem[b])

        def scatter(j, b):
            pltpu.async_copy(rows[b], agg_sh.at[dst_v.at[j]], ssem[b],
                             add=True)

        for b in range(HB):
            gather(b, b)
        for b in range(NBUF):
            drain(rows[b], gsem[b])
            scatter(b, b)
            bn = (b + HB) % NBUF
            if b >= HB:
                drain(rows[bn], ssem[bn])
            gather(b + HB, bn)

        def step(i, carry):
            for b in range(NBUF):
                j = i * NBUF + b
                bn = (b + HB) % NBUF
                drain(rows[b], gsem[b])
                scatter(j, b)
                drain(rows[bn], ssem[bn])
                gather(jnp.minimum(j + HB, CPT - 1), bn)
            return carry

        lax.fori_loop(1, CPT // NBUF, step, 0)
        for b in range(NBUF):
            if b < HB:
                drain(rows[b], gsem[b])
            else:
                drain(rows[b], ssem[b])
        plsc.subcore_barrier()

        for t in range(NRC):
            r0 = s * ROWS_PT + t * RCH
            b = t % 2
            if t >= 2:
                drain(rows[b], gsem[b])
            pltpu.sync_copy(agg_sh.at[pl.ds(r0, RCH)], rows[b])
            pltpu.async_copy(rows[b],
                             out_hbm.at[pl.ds(r0, RCH), pl.ds(col0, HD)],
                             gsem[b])
        for b in range(2):
            drain(rows[b], gsem[b])

    return k(srcp, dst2, xv)


def _zpart(xin, wo, b):
    BR = 2000

    def body(x_ref, wo_ref, b_ref, o_ref):
        o_ref[...] = jnp.dot(x_ref[...], wo_ref[...],
                             preferred_element_type=jnp.float32) + b_ref[...]

    return pl.pallas_call(
        body,
        grid=(N // BR,),
        in_specs=[
            pl.BlockSpec((BR, D), lambda i: (i, 0)),
            pl.BlockSpec((D, D), lambda i: (0, 0)),
            pl.BlockSpec((1, D), lambda i: (0, 0)),
        ],
        out_specs=pl.BlockSpec((BR, D), lambda i: (i, 0)),
        out_shape=jax.ShapeDtypeStruct((N, D), jnp.float32),
    )(xin, wo, b)


def _combine(p, z, xin, wr, relu, residual):
    BR = 2000

    def body(p_ref, z_ref, x_ref, wr_ref, o_ref):
        out = jnp.dot(p_ref[...], wr_ref[...],
                      preferred_element_type=jnp.float32) + z_ref[...]
        if relu:
            out = jnp.maximum(out, 0.0)
        if residual:
            out = out + x_ref[...]
        o_ref[...] = out

    return pl.pallas_call(
        body,
        grid=(N // BR,),
        in_specs=[
            pl.BlockSpec((BR, D), lambda i: (i, 0)),
            pl.BlockSpec((BR, D), lambda i: (i, 0)),
            pl.BlockSpec((BR, D), lambda i: (i, 0)),
            pl.BlockSpec((D, D), lambda i: (0, 0)),
        ],
        out_specs=pl.BlockSpec((BR, D), lambda i: (i, 0)),
        out_shape=jax.ShapeDtypeStruct((N, D), jnp.float32),
    )(p, z, xin, wr)


def kernel(x, edge_index,
           W_rel1, W_root1, b1,
           W_rel2, W_root2, b2,
           W_rel3, W_root3, b3,
           W_rel4, W_root4, b4):
    ei = edge_index.astype(jnp.int32)
    src = ei[0].reshape(E // CHUNK, CHUNK)
    dst2 = ei[1].reshape(E // CHUNK, CHUNK)
    srcp = jnp.stack([2 * src, 2 * src + 1])

    def layer(xin, wr, wo, b, relu, residual):
        p = _segsum_sc(srcp, dst2, xin.reshape(2 * N, HD))
        z = _zpart(xin, wo, b.reshape(1, D))
        return _combine(p, z, xin, wr, relu, residual)

    x1 = layer(x, W_rel1, W_root1, b1, True, False)
    x2 = layer(x1, W_rel2, W_root2, b2, True, True)
    x3 = layer(x2, W_rel3, W_root3, b3, False, True)
    x4 = layer(x3, W_rel4, W_root4, b4, False, True)
    return x4
